# Initial kernel scaffold; baseline (speedup 1.0000x reference)
import jax
import jax.numpy as jnp
from jax.experimental import pallas as pl


def kernel(feat, edge_index, e_feat, W, a, att):
    raise NotImplementedError("write your pallas kernel here")


# speedup vs baseline: 1.0006x; 1.0006x over previous
import jax
import jax.numpy as jnp
import numpy as np
from jax.experimental import pallas as pl

N = 10000
E = 320000
D = 128
RATIO = 0.8
LAMB = 1.0
NEG_SLOPE = 0.2
K = int(np.ceil(RATIO * N))


def _noop_body(x_ref, o_ref):
    o_ref[...] = x_ref[...]


def kernel(feat, edge_index, e_feat, W, a, att):
    src = edge_index[0]
    dst = edge_index[1]
    h = feat @ W
    node_attn = jax.nn.sigmoid(jax.nn.leaky_relu(jnp.squeeze(h @ a, -1), NEG_SLOPE))
    out_deg = jnp.maximum(jnp.zeros((N,), jnp.float32).at[src].add(1.0), 1.0)
    in_deg = jnp.maximum(jnp.zeros((N,), jnp.float32).at[dst].add(1.0), 1.0)
    src_norm = out_deg ** -0.5
    dst_norm = in_deg ** -0.5
    nonself = src != dst
    ew = jnp.where(nonself, e_feat, 0.0)
    msg = (feat * src_norm[:, None])[src] * ew[:, None]
    agg = jnp.zeros((N, D), jnp.float32).at[dst].add(msg)
    fdiff = feat - agg * dst_norm[:, None]
    info_score = jnp.sum(jnp.abs(fdiff), axis=1)
    x_score = info_score * node_attn
    _, perm = jax.lax.top_k(x_score, K)
    feat_p = feat[perm]
    inv = jnp.full((N,), -1, jnp.int32).at[perm].set(jnp.arange(K, dtype=jnp.int32))
    row_m = inv[src]
    col_m = inv[dst]
    valid = (row_m >= 0) & (col_m >= 0)
    row = jnp.where(valid, row_m, 0)
    col = jnp.where(valid, col_m, 0)
    ew2 = jnp.where(valid, e_feat, 0.0)
    loop = jnp.arange(K, dtype=jnp.int32)
    row_all = jnp.concatenate([row, loop])
    col_all = jnp.concatenate([col, loop])
    ew_all = jnp.concatenate([ew2, jnp.ones((K,), jnp.float32)])
    valid_all = jnp.concatenate([valid, jnp.ones((K,), bool)])
    s_src = feat_p @ att[0, :D]
    s_dst = feat_p @ att[0, D:]
    wraw = s_src[row_all] + s_dst[col_all]
    wraw = jax.nn.leaky_relu(wraw, NEG_SLOPE) + ew_all * LAMB
    wraw = jnp.where(valid_all, wraw, -1e9)
    wmax = jax.lax.stop_gradient(jnp.full((K,), -jnp.inf, jnp.float32).at[col_all].max(wraw))
    wexp = jnp.where(valid_all, jnp.exp(wraw - wmax[col_all]), 0.0)
    wsum = jnp.zeros((K,), jnp.float32).at[col_all].add(wexp)
    weights = jnp.where(valid_all, wexp / wsum[col_all], 0.0)
    x_score = pl.pallas_call(
        _noop_body, out_shape=jax.ShapeDtypeStruct(x_score.shape, x_score.dtype)
    )(x_score)
    return (feat_p, weights, perm, x_score)


# speedup vs baseline: 1.1420x; 1.1413x over previous
import dataclasses
import functools

import jax
import jax.numpy as jnp
import numpy as np
from jax import lax
from jax.experimental import pallas as pl
from jax.experimental.pallas import tpu as pltpu
from jax.experimental.pallas import tpu_sc as plsc

N = 10000
E = 320000
D = 128
RATIO = 0.8
LAMB = 1.0
NEG_SLOPE = 0.2
K = int(np.ceil(RATIO * N))

_BLK = 1000

_CHUNK_SIZES = ([10080] * 11 + [9840] * 4 + [9760]) * 2
_BOUNDS = np.concatenate([[0], np.cumsum(_CHUNK_SIZES)]).astype(np.int32)
assert _BOUNDS[-1] == E

_SCAN_W = 2000
_NSCAN = E // _SCAN_W
assert _NSCAN * _SCAN_W == E
_BATCH = 128
_NBATCH = 10240 // _BATCH
_ACC_ROWS = 640
_LIST_CAP = 10240
_HALF = 5008
_GARB = 64
_SH_ROWS = _HALF + _GARB


def _attn_scale_body(feat_ref, w_ref, a_ref, srcn_ref, attn_ref, fs_ref):
    f = feat_ref[...]
    h = jnp.dot(f, w_ref[...])
    t = jnp.dot(h, a_ref[...])
    lr = jnp.where(t >= 0, t, t * NEG_SLOPE)
    attn_ref[...] = jax.nn.sigmoid(lr)
    fs_ref[...] = f * srcn_ref[...]


def _combine_body(vlo_ref, part_ref, out_ref, acc_ref):
    w = pl.program_id(0)

    @pl.when(w == 0)
    def _():
        acc_ref[...] = jnp.zeros_like(acc_ref)

    start = vlo_ref[w]
    acc_ref[pl.ds(start, _ACC_ROWS), :] = (
        acc_ref[pl.ds(start, _ACC_ROWS), :] + part_ref[0])

    @pl.when(w == 31)
    def _():
        out_ref[...] = acc_ref[pl.ds(0, N), :]


def _combine_tc(vlo32, parts):
    return pl.pallas_call(
        _combine_body,
        grid=(32,),
        in_specs=[
            pl.BlockSpec(memory_space=pltpu.SMEM),
            pl.BlockSpec((1, _ACC_ROWS, D), lambda w: (w, 0, 0)),
        ],
        out_specs=pl.BlockSpec((N, D), lambda w: (0, 0)),
        out_shape=jax.ShapeDtypeStruct((N, D), jnp.float32),
        scratch_shapes=[pltpu.VMEM((N + _ACC_ROWS, D), jnp.float32)],
    )(vlo32, parts)


def _score_body(feat_ref, agg_ref, dstn_ref, attn_ref, out_ref):
    fd = jnp.abs(feat_ref[...] - agg_ref[...] * dstn_ref[...])
    acc = fd[:, 0:8]
    for i in range(1, 16):
        acc = acc + fd[:, 8 * i:8 * i + 8]
    a4 = acc[:, 0:4] + acc[:, 4:8]
    a2 = a4[:, 0:2] + a4[:, 2:4]
    a1 = a2[:, 0:1] + a2[:, 1:2]
    out_ref[...] = a1 * attn_ref[...]


def _agg_sc_kernel(dst_h, src_h, ef_h, fs_h, wp_h, out_h,
                   dwin, ids, dvals, idsb, srcs, dstv, efv, ewm, rows, acc,
                   wprm, sem):
    cid = lax.axis_index("c")
    sid = lax.axis_index("s")
    wid = sid * 2 + cid

    pltpu.sync_copy(wp_h.at[pl.ds(pl.multiple_of(wid * 16, 16), 16)], wprm)
    w16 = wprm[...]
    vlo = w16[0]
    vhi = w16[1]
    skip_lo = w16[2]
    take_hi = w16[3]

    z16 = jnp.zeros((16,), jnp.float32)

    @pl.loop(0, _ACC_ROWS)
    def _(r):
        for g in range(8):
            acc[r, pl.ds(g * 16, 16)] = z16

    zi16 = jnp.zeros((16,), jnp.int32)

    @pl.loop(0, _LIST_CAP // 16)
    def _(r):
        ids[pl.ds(r * 16, 16)] = zi16
        dvals[pl.ds(r * 16, 16)] = zi16

    lane = lax.iota(jnp.int32, 16)

    def scan_win(win, carry):
        cnt_lo, cnt_hi, n_own = carry
        pltpu.sync_copy(dst_h.at[pl.ds(pl.multiple_of(win * _SCAN_W, 16), _SCAN_W)], dwin)

        def scan_chunk(c, carry2):
            cnt_lo, cnt_hi, n_own = carry2
            one16 = jnp.ones((16,), jnp.int32)
            zero16 = jnp.zeros((16,), jnp.int32)
            d16 = dwin[pl.ds(pl.multiple_of(c * 16, 16), 16)]
            m_lo = d16 == vlo
            m_hi = d16 == vhi
            cs_lo = plsc.cumsum(jnp.where(m_lo, one16, zero16))
            cs_hi = plsc.cumsum(jnp.where(m_hi, one16, zero16))
            occ_lo = cnt_lo + cs_lo - 1
            occ_hi = cnt_hi + cs_hi - 1
            in_rng = (d16 >= vlo) & (d16 <= vhi)
            ok_lo = jnp.logical_not(m_lo) | (occ_lo >= skip_lo)
            ok_hi = jnp.logical_not(m_hi) | (occ_hi < take_hi)
            owned = in_rng & ok_lo & ok_hi
            cum = plsc.cumsum(jnp.where(owned, one16, zero16))
            addr = n_own + cum - 1
            eids = win * _SCAN_W + c * 16 + lane
            plsc.store_scatter(ids, [addr], eids, mask=owned)
            plsc.store_scatter(dvals, [addr], d16, mask=owned)
            return (cnt_lo + cs_lo[15], cnt_hi + cs_hi[15], n_own + cum[15])

        return lax.fori_loop(0, _SCAN_W // 16, scan_chunk, (cnt_lo, cnt_hi, n_own))

    _, _, n_own = lax.fori_loop(0, _NSCAN, scan_win,
                                (jnp.int32(0), jnp.int32(0), jnp.int32(0)))

    def do_batch(b, _):
        base = b * _BATCH
        nb = jnp.minimum(n_own - base, _BATCH)

        @pl.when(nb > 0)
        def _():
            for g in range(_BATCH // 16):
                off = pl.multiple_of(base + g * 16, 16)
                idsb[pl.ds(g * 16, 16)] = ids[pl.ds(off, 16)]
                dstv[pl.ds(g * 16, 16)] = dvals[pl.ds(off, 16)]
            pltpu.sync_copy(src_h.at[idsb], srcs)
            pltpu.sync_copy(ef_h.at[idsb], efv)
            pltpu.async_copy(fs_h.at[srcs], rows, sem).wait()
            for g in range(_BATCH // 16):
                sl = pl.ds(g * 16, 16)
                ewm[sl] = jnp.where(srcs[sl] == dstv[sl], 0.0, efv[sl])

            if False:
                pass
            else:
                @pl.loop(0, _BATCH // 16)
                def _(g):
                    goff = pl.multiple_of(g * 16, 16)
                    d16 = dstv[pl.ds(goff, 16)]
                    ew16 = ewm[pl.ds(goff, 16)]
                    for l in range(16):
                        j = g * 16 + l

                        @pl.when(j < nb)
                        def _():
                            s = jnp.minimum(d16[l] - vlo, _ACC_ROWS - 1)
                            w = ew16[l]
                            for seg in range(8):
                                sl = pl.ds(seg * 16, 16)
                                acc[s, sl] = acc[s, sl] + rows[j, sl] * w
        return 0

    lax.fori_loop(0, _NBATCH, do_batch, 0)

    pltpu.sync_copy(acc, out_h.at[wid])


def _sc_compiler_params():
    cp = pltpu.CompilerParams()
    if "needs_layout_passes" in pltpu.CompilerParams.__dataclass_fields__:
        cp = dataclasses.replace(cp, needs_layout_passes=False)
    return cp


def _agg_sc(dst, src, e_feat, feat_scaled, wparams):
    mesh = plsc.VectorSubcoreMesh(core_axis_name="c", subcore_axis_name="s")
    kern = pl.kernel(
        _agg_sc_kernel,
        out_type=jax.ShapeDtypeStruct((32, _ACC_ROWS, D), jnp.float32),
        mesh=mesh,
        compiler_params=_sc_compiler_params(),
        scratch_types=[
            pltpu.VMEM((_SCAN_W,), jnp.int32),
            pltpu.VMEM((_LIST_CAP,), jnp.int32),
            pltpu.VMEM((_LIST_CAP,), jnp.int32),
            pltpu.VMEM((_BATCH,), jnp.int32),
            pltpu.VMEM((_BATCH,), jnp.int32),
            pltpu.VMEM((_BATCH,), jnp.int32),
            pltpu.VMEM((_BATCH,), jnp.float32),
            pltpu.VMEM((_BATCH,), jnp.float32),
            pltpu.VMEM((_BATCH, D), jnp.float32),
            pltpu.VMEM((_ACC_ROWS, D), jnp.float32),
            pltpu.VMEM((16,), jnp.int32),
            pltpu.SemaphoreType.DMA,
        ],
    )
    return kern(dst, src, e_feat, feat_scaled, wparams.reshape(-1))


def _worker_params(in_deg_counts):
    r_incl = jnp.cumsum(in_deg_counts.astype(jnp.int32))
    r_excl = r_incl - in_deg_counts.astype(jnp.int32)
    b = jnp.asarray(_BOUNDS)
    blo = b[:32]
    bhi = b[1:33]
    vlo = jnp.searchsorted(r_incl, blo, side="right").astype(jnp.int32)
    vhi = jnp.searchsorted(r_incl, bhi - 1, side="right").astype(jnp.int32)
    skip_lo = blo - r_excl[vlo]
    take_hi = bhi - r_excl[vhi]
    zeros = jnp.zeros((32,), jnp.int32)
    return jnp.stack([vlo, vhi, skip_lo.astype(jnp.int32),
                      take_hi.astype(jnp.int32)] + [zeros] * 12, axis=1)


def kernel(feat, edge_index, e_feat, W, a, att):
    src = edge_index[0]
    dst = edge_index[1]
    in_deg_counts = jnp.zeros((N,), jnp.float32).at[dst].add(1.0)
    out_deg = jnp.maximum(jnp.zeros((N,), jnp.float32).at[src].add(1.0), 1.0)
    in_deg = jnp.maximum(in_deg_counts, 1.0)
    src_norm = jax.lax.rsqrt(out_deg)
    dst_norm = jax.lax.rsqrt(in_deg)

    node_attn2, feat_scaled = pl.pallas_call(
        _attn_scale_body,
        grid=(N // _BLK,),
        in_specs=[
            pl.BlockSpec((_BLK, D), lambda i: (i, 0)),
            pl.BlockSpec((D, D), lambda i: (0, 0)),
            pl.BlockSpec((D, 1), lambda i: (0, 0)),
            pl.BlockSpec((_BLK, 1), lambda i: (i, 0)),
        ],
        out_specs=[
            pl.BlockSpec((_BLK, 1), lambda i: (i, 0)),
            pl.BlockSpec((_BLK, D), lambda i: (i, 0)),
        ],
        out_shape=[
            jax.ShapeDtypeStruct((N, 1), jnp.float32),
            jax.ShapeDtypeStruct((N, D), jnp.float32),
        ],
    )(feat, W, a, src_norm[:, None])

    wparams = _worker_params(in_deg_counts)
    parts = _agg_sc(dst, src, e_feat, feat_scaled, wparams)
    agg = _combine_tc(wparams[:, 0], parts)

    x_score2 = pl.pallas_call(
        _score_body,
        grid=(N // _BLK,),
        in_specs=[
            pl.BlockSpec((_BLK, D), lambda i: (i, 0)),
            pl.BlockSpec((_BLK, D), lambda i: (i, 0)),
            pl.BlockSpec((_BLK, 1), lambda i: (i, 0)),
            pl.BlockSpec((_BLK, 1), lambda i: (i, 0)),
        ],
        out_specs=pl.BlockSpec((_BLK, 1), lambda i: (i, 0)),
        out_shape=jax.ShapeDtypeStruct((N, 1), jnp.float32),
    )(feat, agg, dst_norm[:, None], node_attn2)
    x_score = x_score2[:, 0]

    _, perm = jax.lax.top_k(x_score, K)
    feat_p = feat[perm]
    inv = jnp.full((N,), -1, jnp.int32).at[perm].set(jnp.arange(K, dtype=jnp.int32))
    row_m = inv[src]
    col_m = inv[dst]
    valid = (row_m >= 0) & (col_m >= 0)
    row = jnp.where(valid, row_m, 0)
    col = jnp.where(valid, col_m, 0)
    ew2 = jnp.where(valid, e_feat, 0.0)
    loop = jnp.arange(K, dtype=jnp.int32)
    row_all = jnp.concatenate([row, loop])
    col_all = jnp.concatenate([col, loop])
    ew_all = jnp.concatenate([ew2, jnp.ones((K,), jnp.float32)])
    valid_all = jnp.concatenate([valid, jnp.ones((K,), bool)])
    s_src = feat_p @ att[0, :D]
    s_dst = feat_p @ att[0, D:]
    wraw = s_src[row_all] + s_dst[col_all]
    wraw = jax.nn.leaky_relu(wraw, NEG_SLOPE) + ew_all * LAMB
    wraw = jnp.where(valid_all, wraw, -1e9)
    wmax = jnp.full((K,), -jnp.inf, jnp.float32).at[col_all].max(wraw)
    wexp = jnp.where(valid_all, jnp.exp(wraw - wmax[col_all]), 0.0)
    wsum = jnp.zeros((K,), jnp.float32).at[col_all].add(wexp)
    weights = jnp.where(valid_all, wexp / wsum[col_all], 0.0)
    return (feat_p, weights, perm, x_score)


# speedup vs baseline: 7.3605x; 6.4453x over previous
import dataclasses
import functools

import jax
import jax.numpy as jnp
import numpy as np
from jax import lax
from jax.experimental import pallas as pl
from jax.experimental.pallas import tpu as pltpu
from jax.experimental.pallas import tpu_sc as plsc

N = 10000
E = 320000
D = 128
RATIO = 0.8
LAMB = 1.0
NEG_SLOPE = 0.2
K = int(np.ceil(RATIO * N))

_BLK = 1000

_CHUNK_SIZES = ([10080] * 11 + [9840] * 4 + [9760]) * 2
_BOUNDS = np.concatenate([[0], np.cumsum(_CHUNK_SIZES)]).astype(np.int32)
assert _BOUNDS[-1] == E

_SCAN_W = 2000
_NSCAN = E // _SCAN_W
assert _NSCAN * _SCAN_W == E
_BATCH = 128
_NBATCH = 10240 // _BATCH
_ACC_ROWS = 640
_LIST_CAP = 10240
_HALF = 5008
_GARB = 64
_SH_ROWS = _HALF + _GARB


def _attn_scale_body(feat_ref, w_ref, a_ref, srcn_ref, attl_ref, attr_ref,
                     attn_ref, fs_ref, s_ref, t_ref):
    f = feat_ref[...]
    h = jnp.dot(f, w_ref[...])
    t = jnp.dot(h, a_ref[...])
    lr = jnp.where(t >= 0, t, t * NEG_SLOPE)
    attn_ref[...] = jax.nn.sigmoid(lr)
    fs_ref[...] = f * srcn_ref[...]
    s_ref[...] = jnp.dot(f, attl_ref[...])
    t_ref[...] = jnp.dot(f, attr_ref[...])


def _combine_body(vlo_ref, part_ref, out_ref, acc_ref):
    w = pl.program_id(0)

    @pl.when(w == 0)
    def _():
        acc_ref[...] = jnp.zeros_like(acc_ref)

    start = vlo_ref[w]
    acc_ref[pl.ds(start, _ACC_ROWS), :] = (
        acc_ref[pl.ds(start, _ACC_ROWS), :] + part_ref[0])

    @pl.when(w == 31)
    def _():
        out_ref[...] = acc_ref[pl.ds(0, N), :]


def _combine_tc(vlo32, parts):
    return pl.pallas_call(
        _combine_body,
        grid=(32,),
        in_specs=[
            pl.BlockSpec(memory_space=pltpu.SMEM),
            pl.BlockSpec((1, _ACC_ROWS, D), lambda w: (w, 0, 0)),
        ],
        out_specs=pl.BlockSpec((N, D), lambda w: (0, 0)),
        out_shape=jax.ShapeDtypeStruct((N, D), jnp.float32),
        scratch_shapes=[pltpu.VMEM((N + _ACC_ROWS, D), jnp.float32)],
    )(vlo32, parts)


def _score_body(feat_ref, agg_ref, dstn_ref, attn_ref, out_ref):
    fd = jnp.abs(feat_ref[...] - agg_ref[...] * dstn_ref[...])
    acc = fd[:, 0:8]
    for i in range(1, 16):
        acc = acc + fd[:, 8 * i:8 * i + 8]
    a4 = acc[:, 0:4] + acc[:, 4:8]
    a2 = a4[:, 0:2] + a4[:, 2:4]
    a1 = a2[:, 0:1] + a2[:, 1:2]
    out_ref[...] = a1 * attn_ref[...]


def _agg_sc_kernel(dst_h, src_h, ef_h, fs_h, wp_h, out_h,
                   dwin, ids, dvals, idsb, srcs, dstv, efv, ewm, rows, acc,
                   wprm, sem):
    cid = lax.axis_index("c")
    sid = lax.axis_index("s")
    wid = sid * 2 + cid

    pltpu.sync_copy(wp_h.at[pl.ds(pl.multiple_of(wid * 16, 16), 16)], wprm)
    w16 = wprm[...]
    vlo = w16[0]
    vhi = w16[1]
    skip_lo = w16[2]
    take_hi = w16[3]

    z16 = jnp.zeros((16,), jnp.float32)

    @pl.loop(0, _ACC_ROWS)
    def _(r):
        for g in range(8):
            acc[r, pl.ds(g * 16, 16)] = z16

    zi16 = jnp.zeros((16,), jnp.int32)

    @pl.loop(0, _LIST_CAP // 16)
    def _(r):
        ids[pl.ds(r * 16, 16)] = zi16
        dvals[pl.ds(r * 16, 16)] = zi16

    lane = lax.iota(jnp.int32, 16)

    def scan_win(win, carry):
        cnt_lo, cnt_hi, n_own = carry
        pltpu.sync_copy(dst_h.at[pl.ds(pl.multiple_of(win * _SCAN_W, 16), _SCAN_W)], dwin)

        def scan_chunk(c, carry2):
            cnt_lo, cnt_hi, n_own = carry2
            one16 = jnp.ones((16,), jnp.int32)
            zero16 = jnp.zeros((16,), jnp.int32)
            d16 = dwin[pl.ds(pl.multiple_of(c * 16, 16), 16)]
            m_lo = d16 == vlo
            m_hi = d16 == vhi
            cs_lo = plsc.cumsum(jnp.where(m_lo, one16, zero16))
            cs_hi = plsc.cumsum(jnp.where(m_hi, one16, zero16))
            occ_lo = cnt_lo + cs_lo - 1
            occ_hi = cnt_hi + cs_hi - 1
            in_rng = (d16 >= vlo) & (d16 <= vhi)
            ok_lo = jnp.logical_not(m_lo) | (occ_lo >= skip_lo)
            ok_hi = jnp.logical_not(m_hi) | (occ_hi < take_hi)
            owned = in_rng & ok_lo & ok_hi
            cum = plsc.cumsum(jnp.where(owned, one16, zero16))
            addr = n_own + cum - 1
            eids = win * _SCAN_W + c * 16 + lane
            plsc.store_scatter(ids, [addr], eids, mask=owned)
            plsc.store_scatter(dvals, [addr], d16, mask=owned)
            return (cnt_lo + cs_lo[15], cnt_hi + cs_hi[15], n_own + cum[15])

        return lax.fori_loop(0, _SCAN_W // 16, scan_chunk, (cnt_lo, cnt_hi, n_own))

    _, _, n_own = lax.fori_loop(0, _NSCAN, scan_win,
                                (jnp.int32(0), jnp.int32(0), jnp.int32(0)))

    def do_batch(b, _):
        base = b * _BATCH
        nb = jnp.minimum(n_own - base, _BATCH)

        @pl.when(nb > 0)
        def _():
            for g in range(_BATCH // 16):
                off = pl.multiple_of(base + g * 16, 16)
                idsb[pl.ds(g * 16, 16)] = ids[pl.ds(off, 16)]
                dstv[pl.ds(g * 16, 16)] = dvals[pl.ds(off, 16)]
            pltpu.sync_copy(src_h.at[idsb], srcs)
            pltpu.sync_copy(ef_h.at[idsb], efv)
            pltpu.async_copy(fs_h.at[srcs], rows, sem).wait()
            for g in range(_BATCH // 16):
                sl = pl.ds(g * 16, 16)
                ewm[sl] = jnp.where(srcs[sl] == dstv[sl], 0.0, efv[sl])

            if False:
                pass
            else:
                @pl.loop(0, _BATCH // 16)
                def _(g):
                    goff = pl.multiple_of(g * 16, 16)
                    d16 = dstv[pl.ds(goff, 16)]
                    ew16 = ewm[pl.ds(goff, 16)]
                    for l in range(16):
                        j = g * 16 + l

                        @pl.when(j < nb)
                        def _():
                            s = jnp.minimum(d16[l] - vlo, _ACC_ROWS - 1)
                            w = ew16[l]
                            for seg in range(8):
                                sl = pl.ds(seg * 16, 16)
                                acc[s, sl] = acc[s, sl] + rows[j, sl] * w
        return 0

    lax.fori_loop(0, _NBATCH, do_batch, 0)

    pltpu.sync_copy(acc, out_h.at[wid])


def _sc_compiler_params():
    cp = pltpu.CompilerParams()
    if "needs_layout_passes" in pltpu.CompilerParams.__dataclass_fields__:
        cp = dataclasses.replace(cp, needs_layout_passes=False)
    return cp


def _agg_sc(dst, src, e_feat, feat_scaled, wparams):
    mesh = plsc.VectorSubcoreMesh(core_axis_name="c", subcore_axis_name="s")
    kern = pl.kernel(
        _agg_sc_kernel,
        out_type=jax.ShapeDtypeStruct((32, _ACC_ROWS, D), jnp.float32),
        mesh=mesh,
        compiler_params=_sc_compiler_params(),
        scratch_types=[
            pltpu.VMEM((_SCAN_W,), jnp.int32),
            pltpu.VMEM((_LIST_CAP,), jnp.int32),
            pltpu.VMEM((_LIST_CAP,), jnp.int32),
            pltpu.VMEM((_BATCH,), jnp.int32),
            pltpu.VMEM((_BATCH,), jnp.int32),
            pltpu.VMEM((_BATCH,), jnp.int32),
            pltpu.VMEM((_BATCH,), jnp.float32),
            pltpu.VMEM((_BATCH,), jnp.float32),
            pltpu.VMEM((_BATCH, D), jnp.float32),
            pltpu.VMEM((_ACC_ROWS, D), jnp.float32),
            pltpu.VMEM((16,), jnp.int32),
            pltpu.SemaphoreType.DMA,
        ],
    )
    return kern(dst, src, e_feat, feat_scaled, wparams.reshape(-1))


_EPW = E // 32
_SLW = 256


def _featp_sc_kernel(feat_h, perm_h, out_h, idxb, rows, sem):
    cid = lax.axis_index("c")
    sid = lax.axis_index("s")
    wid = sid * 2 + cid
    nrows = jnp.where(wid < 31, _SLW, K - 31 * _SLW)
    off = pl.multiple_of(wid * _SLW, 8)

    @pl.when(wid < 31)
    def _():
        pltpu.sync_copy(perm_h.at[pl.ds(off, _SLW)], idxb)
        c1 = pltpu.async_copy(feat_h.at[idxb.at[pl.ds(0, 128)]],
                              rows.at[pl.ds(0, 128)], sem)
        c2 = pltpu.async_copy(feat_h.at[idxb.at[pl.ds(128, 128)]],
                              rows.at[pl.ds(128, 128)], sem)
        c1.wait()
        c2.wait()
        pltpu.sync_copy(rows, out_h.at[pl.ds(off, _SLW)])

    @pl.when(wid == 31)
    def _():
        pltpu.sync_copy(perm_h.at[pl.ds(31 * _SLW, 64)], idxb.at[pl.ds(0, 64)])
        pltpu.async_copy(feat_h.at[idxb.at[pl.ds(0, 64)]],
                         rows.at[pl.ds(0, 64)], sem).wait()
        pltpu.sync_copy(rows.at[pl.ds(0, 64)], out_h.at[pl.ds(31 * _SLW, 64)])


def _featp_sc(feat, perm):
    mesh = plsc.VectorSubcoreMesh(core_axis_name="c", subcore_axis_name="s")
    return pl.kernel(
        _featp_sc_kernel,
        out_type=jax.ShapeDtypeStruct((K, D), jnp.float32),
        mesh=mesh,
        compiler_params=_sc_compiler_params(),
        scratch_types=[
            pltpu.VMEM((_SLW,), jnp.int32),
            pltpu.VMEM((_SLW, D), jnp.float32),
            pltpu.SemaphoreType.DMA,
        ],
    )(feat, perm)


_TW = 2000


def _lsum_rmw(lsum, d16, v16, lane):
    for l in range(16):
        d = d16[l]
        row16 = pl.multiple_of((d >> 4) * 16, 16)
        cur = lsum[pl.ds(row16, 16)]
        add = jnp.where(lane == (d & 15), v16[l], 0.0)
        lsum[pl.ds(row16, 16)] = cur + add


def _edge1_sc_kernel(src_h, dst_h, ef_h, s_h, t_h, perm_h,
                     wexp_h, lsum_h,
                     sbuf, tbuf, sel, lsum, permb, swin, dwin, efwin, wwin,
                     sem):
    cid = lax.axis_index("c")
    sid = lax.axis_index("s")
    wid = sid * 2 + cid
    lane = lax.iota(jnp.int32, 16)
    zf16 = jnp.zeros((16,), jnp.float32)
    zi16 = jnp.zeros((16,), jnp.int32)
    one16 = jnp.full((16,), 1, jnp.int32)

    pltpu.sync_copy(s_h, sbuf)
    pltpu.sync_copy(t_h, tbuf)
    pltpu.sync_copy(perm_h, permb)

    @pl.loop(0, N // 16)
    def _(r):
        roff = pl.multiple_of(r * 16, 16)
        sel[pl.ds(roff, 16)] = zi16
        lsum[pl.ds(roff, 16)] = zf16

    @pl.loop(0, K // 16)
    def _(r):
        roff = pl.multiple_of(r * 16, 16)
        p16 = permb[pl.ds(roff, 16)]
        plsc.store_scatter(sel, [p16], one16)

    nself = jnp.where(wid < 31, _SLW, K - 31 * _SLW)

    @pl.loop(0, _SLW // 16)
    def _(c):
        @pl.when(c * 16 < nself)
        def _():
            poff = pl.multiple_of(wid * _SLW + c * 16, 16)
            p16 = permb[pl.ds(poff, 16)]
            sv = plsc.load_gather(sbuf, [p16])
            tv = plsc.load_gather(tbuf, [p16])
            x = sv + tv
            wraw = jnp.where(x >= 0, x, x * NEG_SLOPE) + 1.0
            we = jnp.exp(wraw)
            wwin[pl.ds(pl.multiple_of(c * 16, 16), 16)] = we
            _lsum_rmw(lsum, p16, we, lane)

    @pl.when(wid < 31)
    def _():
        pltpu.sync_copy(wwin.at[pl.ds(0, _SLW)],
                        wexp_h.at[pl.ds(E + wid * _SLW, _SLW)])

    @pl.when(wid == 31)
    def _():
        pltpu.sync_copy(wwin.at[pl.ds(0, 64)],
                        wexp_h.at[pl.ds(E + 31 * _SLW, 64)])

    @pl.loop(0, _EPW // _TW)
    def _(win):
        base = pl.multiple_of(wid * _EPW + win * _TW, 16)
        pltpu.sync_copy(src_h.at[pl.ds(base, _TW)], swin)
        pltpu.sync_copy(dst_h.at[pl.ds(base, _TW)], dwin)
        pltpu.sync_copy(ef_h.at[pl.ds(base, _TW)], efwin)

        @pl.loop(0, _TW // 16)
        def _(c):
            coff = pl.multiple_of(c * 16, 16)
            s16 = swin[pl.ds(coff, 16)]
            d16 = dwin[pl.ds(coff, 16)]
            e16 = efwin[pl.ds(coff, 16)]
            sv = plsc.load_gather(sbuf, [s16])
            tv = plsc.load_gather(tbuf, [d16])
            vs = plsc.load_gather(sel, [s16])
            vd = plsc.load_gather(sel, [d16])
            valid = (vs > 0) & (vd > 0)
            x = sv + tv
            wraw = jnp.where(x >= 0, x, x * NEG_SLOPE) + e16 * LAMB
            we = jnp.where(valid, jnp.exp(wraw), zf16)
            wwin[pl.ds(coff, 16)] = we
            _lsum_rmw(lsum, d16, we, lane)

        pltpu.sync_copy(wwin.at[pl.ds(0, _TW)], wexp_h.at[pl.ds(base, _TW)])

    pltpu.sync_copy(lsum, lsum_h.at[wid])


def _edge1_sc(src, dst, e_feat, s_full, t_full, perm):
    mesh = plsc.VectorSubcoreMesh(core_axis_name="c", subcore_axis_name="s")
    return pl.kernel(
        _edge1_sc_kernel,
        out_type=[
            jax.ShapeDtypeStruct((E + K,), jnp.float32),
            jax.ShapeDtypeStruct((32, N), jnp.float32),
        ],
        mesh=mesh,
        compiler_params=_sc_compiler_params(),
        scratch_types=[
            pltpu.VMEM((N,), jnp.float32),
            pltpu.VMEM((N,), jnp.float32),
            pltpu.VMEM((N,), jnp.int32),
            pltpu.VMEM((N,), jnp.float32),
            pltpu.VMEM((K,), jnp.int32),
            pltpu.VMEM((_TW,), jnp.int32),
            pltpu.VMEM((_TW,), jnp.int32),
            pltpu.VMEM((_TW,), jnp.float32),
            pltpu.VMEM((_TW,), jnp.float32),
            pltpu.SemaphoreType.DMA,
        ],
    )(src, dst, e_feat, s_full, t_full, perm)


def _wsum_body(parts_ref, out_ref):
    out_ref[...] = jnp.sum(parts_ref[...], axis=0, keepdims=True)


def _edge2_sc_kernel(dst_h, wexp_h, wsum_h, perm_h, out_h,
                     wsbuf, permb, dwin, wwin, owin, sem):
    cid = lax.axis_index("c")
    sid = lax.axis_index("s")
    wid = sid * 2 + cid

    pltpu.sync_copy(wsum_h, wsbuf)
    pltpu.sync_copy(perm_h, permb)

    @pl.loop(0, _EPW // _TW)
    def _(win):
        base = pl.multiple_of(wid * _EPW + win * _TW, 16)
        pltpu.sync_copy(dst_h.at[pl.ds(base, _TW)], dwin)
        pltpu.sync_copy(wexp_h.at[pl.ds(base, _TW)], wwin)

        @pl.loop(0, _TW // 16)
        def _(c):
            coff = pl.multiple_of(c * 16, 16)
            d16 = dwin[pl.ds(coff, 16)]
            we = wwin[pl.ds(coff, 16)]
            dsum = plsc.load_gather(wsbuf, [d16])
            den = jnp.where(dsum > 0, dsum, jnp.ones((16,), jnp.float32))
            owin[pl.ds(coff, 16)] = we / den

        pltpu.sync_copy(owin.at[pl.ds(0, _TW)], out_h.at[pl.ds(base, _TW)])

    nself = jnp.where(wid < 31, _SLW, K - 31 * _SLW)

    @pl.when(wid < 31)
    def _():
        pltpu.sync_copy(wexp_h.at[pl.ds(E + wid * _SLW, _SLW)],
                        wwin.at[pl.ds(0, _SLW)])

    @pl.when(wid == 31)
    def _():
        pltpu.sync_copy(wexp_h.at[pl.ds(E + 31 * _SLW, 64)],
                        wwin.at[pl.ds(0, 64)])

    @pl.loop(0, _SLW // 16)
    def _(c):
        @pl.when(c * 16 < nself)
        def _():
            poff = pl.multiple_of(wid * _SLW + c * 16, 16)
            coff = pl.multiple_of(c * 16, 16)
            p16 = permb[pl.ds(poff, 16)]
            we = wwin[pl.ds(coff, 16)]
            dsum = plsc.load_gather(wsbuf, [p16])
            den = jnp.where(dsum > 0, dsum, jnp.ones((16,), jnp.float32))
            owin[pl.ds(coff, 16)] = we / den

    @pl.when(wid < 31)
    def _():
        pltpu.sync_copy(owin.at[pl.ds(0, _SLW)],
                        out_h.at[pl.ds(E + wid * _SLW, _SLW)])

    @pl.when(wid == 31)
    def _():
        pltpu.sync_copy(owin.at[pl.ds(0, 64)],
                        out_h.at[pl.ds(E + 31 * _SLW, 64)])


def _edge2_sc(dst, wexp_all, wsumf, perm):
    mesh = plsc.VectorSubcoreMesh(core_axis_name="c", subcore_axis_name="s")
    return pl.kernel(
        _edge2_sc_kernel,
        out_type=jax.ShapeDtypeStruct((E + K,), jnp.float32),
        mesh=mesh,
        compiler_params=_sc_compiler_params(),
        scratch_types=[
            pltpu.VMEM((N,), jnp.float32),
            pltpu.VMEM((K,), jnp.int32),
            pltpu.VMEM((_TW,), jnp.int32),
            pltpu.VMEM((_TW,), jnp.float32),
            pltpu.VMEM((_TW,), jnp.float32),
            pltpu.SemaphoreType.DMA,
        ],
    )(dst, wexp_all, wsumf, perm)


def _worker_params(in_deg_counts):
    r_incl = jnp.cumsum(in_deg_counts.astype(jnp.int32))
    r_excl = r_incl - in_deg_counts.astype(jnp.int32)
    b = jnp.asarray(_BOUNDS)
    blo = b[:32]
    bhi = b[1:33]
    vlo = jnp.searchsorted(r_incl, blo, side="right").astype(jnp.int32)
    vhi = jnp.searchsorted(r_incl, bhi - 1, side="right").astype(jnp.int32)
    skip_lo = blo - r_excl[vlo]
    take_hi = bhi - r_excl[vhi]
    zeros = jnp.zeros((32,), jnp.int32)
    return jnp.stack([vlo, vhi, skip_lo.astype(jnp.int32),
                      take_hi.astype(jnp.int32)] + [zeros] * 12, axis=1)


def kernel(feat, edge_index, e_feat, W, a, att):
    src = edge_index[0]
    dst = edge_index[1]
    in_deg_counts = jnp.zeros((N,), jnp.float32).at[dst].add(1.0)
    out_deg = jnp.maximum(jnp.zeros((N,), jnp.float32).at[src].add(1.0), 1.0)
    in_deg = jnp.maximum(in_deg_counts, 1.0)
    src_norm = jax.lax.rsqrt(out_deg)
    dst_norm = jax.lax.rsqrt(in_deg)

    node_attn2, feat_scaled, s_full2, t_full2 = pl.pallas_call(
        _attn_scale_body,
        grid=(N // _BLK,),
        in_specs=[
            pl.BlockSpec((_BLK, D), lambda i: (i, 0)),
            pl.BlockSpec((D, D), lambda i: (0, 0)),
            pl.BlockSpec((D, 1), lambda i: (0, 0)),
            pl.BlockSpec((_BLK, 1), lambda i: (i, 0)),
            pl.BlockSpec((D, 1), lambda i: (0, 0)),
            pl.BlockSpec((D, 1), lambda i: (0, 0)),
        ],
        out_specs=[
            pl.BlockSpec((_BLK, 1), lambda i: (i, 0)),
            pl.BlockSpec((_BLK, D), lambda i: (i, 0)),
            pl.BlockSpec((_BLK, 1), lambda i: (i, 0)),
            pl.BlockSpec((_BLK, 1), lambda i: (i, 0)),
        ],
        out_shape=[
            jax.ShapeDtypeStruct((N, 1), jnp.float32),
            jax.ShapeDtypeStruct((N, D), jnp.float32),
            jax.ShapeDtypeStruct((N, 1), jnp.float32),
            jax.ShapeDtypeStruct((N, 1), jnp.float32),
        ],
    )(feat, W, a, src_norm[:, None], att[0, :D][:, None], att[0, D:][:, None])

    wparams = _worker_params(in_deg_counts)
    parts = _agg_sc(dst, src, e_feat, feat_scaled, wparams)
    agg = _combine_tc(wparams[:, 0], parts)

    x_score2 = pl.pallas_call(
        _score_body,
        grid=(N // _BLK,),
        in_specs=[
            pl.BlockSpec((_BLK, D), lambda i: (i, 0)),
            pl.BlockSpec((_BLK, D), lambda i: (i, 0)),
            pl.BlockSpec((_BLK, 1), lambda i: (i, 0)),
            pl.BlockSpec((_BLK, 1), lambda i: (i, 0)),
        ],
        out_specs=pl.BlockSpec((_BLK, 1), lambda i: (i, 0)),
        out_shape=jax.ShapeDtypeStruct((N, 1), jnp.float32),
    )(feat, agg, dst_norm[:, None], node_attn2)
    x_score = x_score2[:, 0]

    _, perm = jax.lax.top_k(x_score, K)
    feat_p = _featp_sc(feat, perm)
    wexp_all, lsum_parts = _edge1_sc(src, dst, e_feat,
                                     s_full2[:, 0], t_full2[:, 0], perm)
    wsumf = pl.pallas_call(
        _wsum_body,
        out_shape=jax.ShapeDtypeStruct((1, N), jnp.float32),
    )(lsum_parts)
    weights = _edge2_sc(dst, wexp_all, wsumf[0], perm)
    return (feat_p, weights, perm, x_score)


# speedup vs baseline: 10.3463x; 1.4057x over previous
import dataclasses
import functools

import jax
import jax.numpy as jnp
import numpy as np
from jax import lax
from jax.experimental import pallas as pl
from jax.experimental.pallas import tpu as pltpu
from jax.experimental.pallas import tpu_sc as plsc

N = 10000
E = 320000
D = 128
RATIO = 0.8
LAMB = 1.0
NEG_SLOPE = 0.2
K = int(np.ceil(RATIO * N))

_BLK = 1000

_CHUNK_SIZES = ([10080] * 11 + [9840] * 4 + [9760]) * 2
_BOUNDS = np.concatenate([[0], np.cumsum(_CHUNK_SIZES)]).astype(np.int32)
assert _BOUNDS[-1] == E

_SCAN_W = 2000
_NSCAN = E // _SCAN_W
assert _NSCAN * _SCAN_W == E
_BATCH = 128
_NBATCH = 10240 // _BATCH
_ACC_ROWS = 640
_LIST_CAP = 10240
_HALF = 5008
_GARB = 64
_SH_ROWS = _HALF + _GARB


def _attn_scale_body(feat_ref, w_ref, a_ref, srcn_ref, attl_ref, attr_ref,
                     attn_ref, fs_ref, s_ref, t_ref):
    f = feat_ref[...]
    h = jnp.dot(f, w_ref[...])
    t = jnp.dot(h, a_ref[...])
    lr = jnp.where(t >= 0, t, t * NEG_SLOPE)
    attn_ref[...] = jax.nn.sigmoid(lr)
    fs_ref[...] = f * srcn_ref[...]
    s_ref[...] = jnp.dot(f, attl_ref[...])
    t_ref[...] = jnp.dot(f, attr_ref[...])


def _combine_body(vlo_ref, part_ref, out_ref, acc_ref):
    w = pl.program_id(0)

    @pl.when(w == 0)
    def _():
        acc_ref[...] = jnp.zeros_like(acc_ref)

    start = vlo_ref[w]
    acc_ref[pl.ds(start, _ACC_ROWS), :] = (
        acc_ref[pl.ds(start, _ACC_ROWS), :] + part_ref[0])

    @pl.when(w == 31)
    def _():
        out_ref[...] = acc_ref[pl.ds(0, N), :]


def _combine_tc(vlo32, parts):
    return pl.pallas_call(
        _combine_body,
        grid=(32,),
        in_specs=[
            pl.BlockSpec(memory_space=pltpu.SMEM),
            pl.BlockSpec((1, _ACC_ROWS, D), lambda w: (w, 0, 0)),
        ],
        out_specs=pl.BlockSpec((N, D), lambda w: (0, 0)),
        out_shape=jax.ShapeDtypeStruct((N, D), jnp.float32),
        scratch_shapes=[pltpu.VMEM((N + _ACC_ROWS, D), jnp.float32)],
    )(vlo32, parts)


def _score_body(feat_ref, agg_ref, dstn_ref, attn_ref, out_ref):
    fd = jnp.abs(feat_ref[...] - agg_ref[...] * dstn_ref[...])
    acc = fd[:, 0:8]
    for i in range(1, 16):
        acc = acc + fd[:, 8 * i:8 * i + 8]
    a4 = acc[:, 0:4] + acc[:, 4:8]
    a2 = a4[:, 0:2] + a4[:, 2:4]
    a1 = a2[:, 0:1] + a2[:, 1:2]
    out_ref[...] = a1 * attn_ref[...]


def _agg_sc_kernel(dst_h, src_h, ef_h, fs_h, wp_h, out_h,
                   dwin, ids, dvals, idsb, srcs, dstv, efv, ewm, rows, acc,
                   wprm, sem):
    cid = lax.axis_index("c")
    sid = lax.axis_index("s")
    wid = sid * 2 + cid

    pltpu.sync_copy(wp_h.at[pl.ds(pl.multiple_of(wid * 16, 16), 16)], wprm)
    w16 = wprm[...]
    vlo = w16[0]
    vhi = w16[1]
    skip_lo = w16[2]
    take_hi = w16[3]

    z16 = jnp.zeros((16,), jnp.float32)

    @pl.loop(0, _ACC_ROWS)
    def _(r):
        for g in range(8):
            acc[r, pl.ds(g * 16, 16)] = z16

    zi16 = jnp.zeros((16,), jnp.int32)

    @pl.loop(0, _LIST_CAP // 16)
    def _(r):
        ids[pl.ds(r * 16, 16)] = zi16
        dvals[pl.ds(r * 16, 16)] = zi16

    lane = lax.iota(jnp.int32, 16)

    def scan_win(win, carry):
        cnt_lo, cnt_hi, n_own = carry
        pltpu.sync_copy(dst_h.at[pl.ds(pl.multiple_of(win * _SCAN_W, 16), _SCAN_W)], dwin)

        def scan_chunk(c, carry2):
            cnt_lo, cnt_hi, n_own = carry2
            one16 = jnp.ones((16,), jnp.int32)
            zero16 = jnp.zeros((16,), jnp.int32)
            d16 = dwin[pl.ds(pl.multiple_of(c * 16, 16), 16)]
            m_lo = d16 == vlo
            m_hi = d16 == vhi
            cs_lo = plsc.cumsum(jnp.where(m_lo, one16, zero16))
            cs_hi = plsc.cumsum(jnp.where(m_hi, one16, zero16))
            occ_lo = cnt_lo + cs_lo - 1
            occ_hi = cnt_hi + cs_hi - 1
            in_rng = (d16 >= vlo) & (d16 <= vhi)
            ok_lo = jnp.logical_not(m_lo) | (occ_lo >= skip_lo)
            ok_hi = jnp.logical_not(m_hi) | (occ_hi < take_hi)
            owned = in_rng & ok_lo & ok_hi
            cum = plsc.cumsum(jnp.where(owned, one16, zero16))
            addr = n_own + cum - 1
            eids = win * _SCAN_W + c * 16 + lane
            plsc.store_scatter(ids, [addr], eids, mask=owned)
            plsc.store_scatter(dvals, [addr], d16, mask=owned)
            return (cnt_lo + cs_lo[15], cnt_hi + cs_hi[15], n_own + cum[15])

        return lax.fori_loop(0, _SCAN_W // 16, scan_chunk, (cnt_lo, cnt_hi, n_own))

    _, _, n_own = lax.fori_loop(0, _NSCAN, scan_win,
                                (jnp.int32(0), jnp.int32(0), jnp.int32(0)))

    def do_batch(b, _):
        base = b * _BATCH
        nb = jnp.minimum(n_own - base, _BATCH)

        @pl.when(nb > 0)
        def _():
            for g in range(_BATCH // 16):
                off = pl.multiple_of(base + g * 16, 16)
                idsb[pl.ds(g * 16, 16)] = ids[pl.ds(off, 16)]
                dstv[pl.ds(g * 16, 16)] = dvals[pl.ds(off, 16)]
            pltpu.sync_copy(src_h.at[idsb], srcs)
            pltpu.sync_copy(ef_h.at[idsb], efv)
            pltpu.async_copy(fs_h.at[srcs], rows, sem).wait()
            for g in range(_BATCH // 16):
                sl = pl.ds(g * 16, 16)
                ewm[sl] = jnp.where(srcs[sl] == dstv[sl], 0.0, efv[sl])

            if False:
                pass
            else:
                @pl.loop(0, _BATCH // 16)
                def _(g):
                    goff = pl.multiple_of(g * 16, 16)
                    d16 = dstv[pl.ds(goff, 16)]
                    ew16 = ewm[pl.ds(goff, 16)]
                    for l in range(16):
                        j = g * 16 + l

                        @pl.when(j < nb)
                        def _():
                            s = jnp.minimum(d16[l] - vlo, _ACC_ROWS - 1)
                            w = ew16[l]
                            for seg in range(8):
                                sl = pl.ds(seg * 16, 16)
                                acc[s, sl] = acc[s, sl] + rows[j, sl] * w
        return 0

    lax.fori_loop(0, _NBATCH, do_batch, 0)

    pltpu.sync_copy(acc, out_h.at[wid])


def _sc_compiler_params():
    cp = pltpu.CompilerParams()
    if "needs_layout_passes" in pltpu.CompilerParams.__dataclass_fields__:
        cp = dataclasses.replace(cp, needs_layout_passes=False)
    return cp


def _agg_sc(dst, src, e_feat, feat_scaled, wparams):
    mesh = plsc.VectorSubcoreMesh(core_axis_name="c", subcore_axis_name="s")
    kern = pl.kernel(
        _agg_sc_kernel,
        out_type=jax.ShapeDtypeStruct((32, _ACC_ROWS, D), jnp.float32),
        mesh=mesh,
        compiler_params=_sc_compiler_params(),
        scratch_types=[
            pltpu.VMEM((_SCAN_W,), jnp.int32),
            pltpu.VMEM((_LIST_CAP,), jnp.int32),
            pltpu.VMEM((_LIST_CAP,), jnp.int32),
            pltpu.VMEM((_BATCH,), jnp.int32),
            pltpu.VMEM((_BATCH,), jnp.int32),
            pltpu.VMEM((_BATCH,), jnp.int32),
            pltpu.VMEM((_BATCH,), jnp.float32),
            pltpu.VMEM((_BATCH,), jnp.float32),
            pltpu.VMEM((_BATCH, D), jnp.float32),
            pltpu.VMEM((_ACC_ROWS, D), jnp.float32),
            pltpu.VMEM((16,), jnp.int32),
            pltpu.SemaphoreType.DMA,
        ],
    )
    return kern(dst, src, e_feat, feat_scaled, wparams.reshape(-1))


def _deg_sc_kernel(src_h, dst_h, osrc_h, odst_h, hsrc, hdst, win):
    cid = lax.axis_index("c")
    sid = lax.axis_index("s")
    wid = sid * 2 + cid
    zi16 = jnp.zeros((16,), jnp.int32)

    @pl.loop(0, N // 16)
    def _(r):
        roff = pl.multiple_of(r * 16, 16)
        hsrc[pl.ds(roff, 16)] = zi16
        hdst[pl.ds(roff, 16)] = zi16

    for which, (in_h, hist) in enumerate([(src_h, hsrc), (dst_h, hdst)]):
        @pl.loop(0, (E // 32) // 2000)
        def _(w, in_h=in_h, hist=hist):
            base = pl.multiple_of(wid * (E // 32) + w * 2000, 16)
            pltpu.sync_copy(in_h.at[pl.ds(base, 2000)], win)

            @pl.loop(0, 2000 // 16)
            def _(c, hist=hist):
                coff = pl.multiple_of(c * 16, 16)
                v16 = win[pl.ds(coff, 16)]
                cnt, last = plsc.scan_count(v16)
                plsc.addupdate_scatter(hist, [v16], cnt, mask=last)

    pltpu.sync_copy(hsrc, osrc_h.at[wid])
    pltpu.sync_copy(hdst, odst_h.at[wid])


def _deg_sc(src, dst):
    mesh = plsc.VectorSubcoreMesh(core_axis_name="c", subcore_axis_name="s")
    return pl.kernel(
        _deg_sc_kernel,
        out_type=[
            jax.ShapeDtypeStruct((32, N), jnp.int32),
            jax.ShapeDtypeStruct((32, N), jnp.int32),
        ],
        mesh=mesh,
        compiler_params=_sc_compiler_params(),
        scratch_types=[
            pltpu.VMEM((N,), jnp.int32),
            pltpu.VMEM((N,), jnp.int32),
            pltpu.VMEM((2000,), jnp.int32),
        ],
    )(src, dst)


def _isum_body(p0_ref, p1_ref, o0_ref, o1_ref):
    o0_ref[...] = jnp.sum(p0_ref[...], axis=0, keepdims=True)
    o1_ref[...] = jnp.sum(p1_ref[...], axis=0, keepdims=True)


_EPW = E // 32
_SLW = 256


def _featp_sc_kernel(feat_h, perm_h, out_h, idxb, rows, sem):
    cid = lax.axis_index("c")
    sid = lax.axis_index("s")
    wid = sid * 2 + cid
    nrows = jnp.where(wid < 31, _SLW, K - 31 * _SLW)
    off = pl.multiple_of(wid * _SLW, 8)

    @pl.when(wid < 31)
    def _():
        pltpu.sync_copy(perm_h.at[pl.ds(off, _SLW)], idxb)
        c1 = pltpu.async_copy(feat_h.at[idxb.at[pl.ds(0, 128)]],
                              rows.at[pl.ds(0, 128)], sem)
        c2 = pltpu.async_copy(feat_h.at[idxb.at[pl.ds(128, 128)]],
                              rows.at[pl.ds(128, 128)], sem)
        c1.wait()
        c2.wait()
        pltpu.sync_copy(rows, out_h.at[pl.ds(off, _SLW)])

    @pl.when(wid == 31)
    def _():
        pltpu.sync_copy(perm_h.at[pl.ds(31 * _SLW, 64)], idxb.at[pl.ds(0, 64)])
        pltpu.async_copy(feat_h.at[idxb.at[pl.ds(0, 64)]],
                         rows.at[pl.ds(0, 64)], sem).wait()
        pltpu.sync_copy(rows.at[pl.ds(0, 64)], out_h.at[pl.ds(31 * _SLW, 64)])


def _featp_sc(feat, perm):
    mesh = plsc.VectorSubcoreMesh(core_axis_name="c", subcore_axis_name="s")
    return pl.kernel(
        _featp_sc_kernel,
        out_type=jax.ShapeDtypeStruct((K, D), jnp.float32),
        mesh=mesh,
        compiler_params=_sc_compiler_params(),
        scratch_types=[
            pltpu.VMEM((_SLW,), jnp.int32),
            pltpu.VMEM((_SLW, D), jnp.float32),
            pltpu.SemaphoreType.DMA,
        ],
    )(feat, perm)


_TW = 2000


def _lsum_rmw(lsum, d16, v16, lane):
    for l in range(16):
        d = d16[l]
        row16 = pl.multiple_of((d >> 4) * 16, 16)
        cur = lsum[pl.ds(row16, 16)]
        add = jnp.where(lane == (d & 15), v16[l], 0.0)
        lsum[pl.ds(row16, 16)] = cur + add


def _edge1_sc_kernel(src_h, dst_h, ef_h, s_h, t_h, perm_h,
                     wexp_h, lsum_h,
                     sbuf, tbuf, sel, lsum, permb, swin, dwin, efwin, wwin,
                     sem):
    cid = lax.axis_index("c")
    sid = lax.axis_index("s")
    wid = sid * 2 + cid
    lane = lax.iota(jnp.int32, 16)
    zf16 = jnp.zeros((16,), jnp.float32)
    zi16 = jnp.zeros((16,), jnp.int32)
    one16 = jnp.full((16,), 1, jnp.int32)

    pltpu.sync_copy(s_h, sbuf)
    pltpu.sync_copy(t_h, tbuf)
    pltpu.sync_copy(perm_h, permb)

    @pl.loop(0, N // 16)
    def _(r):
        roff = pl.multiple_of(r * 16, 16)
        sel[pl.ds(roff, 16)] = zi16
        lsum[pl.ds(roff, 16)] = zf16

    @pl.loop(0, K // 16)
    def _(r):
        roff = pl.multiple_of(r * 16, 16)
        p16 = permb[pl.ds(roff, 16)]
        plsc.store_scatter(sel, [p16], one16)

    nself = jnp.where(wid < 31, _SLW, K - 31 * _SLW)

    @pl.loop(0, _SLW // 16)
    def _(c):
        @pl.when(c * 16 < nself)
        def _():
            poff = pl.multiple_of(wid * _SLW + c * 16, 16)
            p16 = permb[pl.ds(poff, 16)]
            sv = plsc.load_gather(sbuf, [p16])
            tv = plsc.load_gather(tbuf, [p16])
            x = sv + tv
            wraw = jnp.where(x >= 0, x, x * NEG_SLOPE) + 1.0
            we = jnp.exp(wraw)
            wwin[pl.ds(pl.multiple_of(c * 16, 16), 16)] = we
            _lsum_rmw(lsum, p16, we, lane)

    @pl.when(wid < 31)
    def _():
        pltpu.sync_copy(wwin.at[pl.ds(0, _SLW)],
                        wexp_h.at[pl.ds(E + wid * _SLW, _SLW)])

    @pl.when(wid == 31)
    def _():
        pltpu.sync_copy(wwin.at[pl.ds(0, 64)],
                        wexp_h.at[pl.ds(E + 31 * _SLW, 64)])

    @pl.loop(0, _EPW // _TW)
    def _(win):
        base = pl.multiple_of(wid * _EPW + win * _TW, 16)
        pltpu.sync_copy(src_h.at[pl.ds(base, _TW)], swin)
        pltpu.sync_copy(dst_h.at[pl.ds(base, _TW)], dwin)
        pltpu.sync_copy(ef_h.at[pl.ds(base, _TW)], efwin)

        @pl.loop(0, _TW // 16)
        def _(c):
            coff = pl.multiple_of(c * 16, 16)
            s16 = swin[pl.ds(coff, 16)]
            d16 = dwin[pl.ds(coff, 16)]
            e16 = efwin[pl.ds(coff, 16)]
            sv = plsc.load_gather(sbuf, [s16])
            tv = plsc.load_gather(tbuf, [d16])
            vs = plsc.load_gather(sel, [s16])
            vd = plsc.load_gather(sel, [d16])
            valid = (vs > 0) & (vd > 0)
            x = sv + tv
            wraw = jnp.where(x >= 0, x, x * NEG_SLOPE) + e16 * LAMB
            we = jnp.where(valid, jnp.exp(wraw), zf16)
            wwin[pl.ds(coff, 16)] = we
            _lsum_rmw(lsum, d16, we, lane)

        pltpu.sync_copy(wwin.at[pl.ds(0, _TW)], wexp_h.at[pl.ds(base, _TW)])

    pltpu.sync_copy(lsum, lsum_h.at[wid])


def _edge1_sc(src, dst, e_feat, s_full, t_full, perm):
    mesh = plsc.VectorSubcoreMesh(core_axis_name="c", subcore_axis_name="s")
    return pl.kernel(
        _edge1_sc_kernel,
        out_type=[
            jax.ShapeDtypeStruct((E + K,), jnp.float32),
            jax.ShapeDtypeStruct((32, N), jnp.float32),
        ],
        mesh=mesh,
        compiler_params=_sc_compiler_params(),
        scratch_types=[
            pltpu.VMEM((N,), jnp.float32),
            pltpu.VMEM((N,), jnp.float32),
            pltpu.VMEM((N,), jnp.int32),
            pltpu.VMEM((N,), jnp.float32),
            pltpu.VMEM((K,), jnp.int32),
            pltpu.VMEM((_TW,), jnp.int32),
            pltpu.VMEM((_TW,), jnp.int32),
            pltpu.VMEM((_TW,), jnp.float32),
            pltpu.VMEM((_TW,), jnp.float32),
            pltpu.SemaphoreType.DMA,
        ],
    )(src, dst, e_feat, s_full, t_full, perm)


def _wsum_body(parts_ref, out_ref):
    out_ref[...] = jnp.sum(parts_ref[...], axis=0, keepdims=True)


def _edge2_sc_kernel(dst_h, wexp_h, wsum_h, perm_h, out_h,
                     wsbuf, permb, dwin, wwin, owin, sem):
    cid = lax.axis_index("c")
    sid = lax.axis_index("s")
    wid = sid * 2 + cid

    pltpu.sync_copy(wsum_h, wsbuf)
    pltpu.sync_copy(perm_h, permb)

    @pl.loop(0, _EPW // _TW)
    def _(win):
        base = pl.multiple_of(wid * _EPW + win * _TW, 16)
        pltpu.sync_copy(dst_h.at[pl.ds(base, _TW)], dwin)
        pltpu.sync_copy(wexp_h.at[pl.ds(base, _TW)], wwin)

        @pl.loop(0, _TW // 16)
        def _(c):
            coff = pl.multiple_of(c * 16, 16)
            d16 = dwin[pl.ds(coff, 16)]
            we = wwin[pl.ds(coff, 16)]
            dsum = plsc.load_gather(wsbuf, [d16])
            den = jnp.where(dsum > 0, dsum, jnp.ones((16,), jnp.float32))
            owin[pl.ds(coff, 16)] = we / den

        pltpu.sync_copy(owin.at[pl.ds(0, _TW)], out_h.at[pl.ds(base, _TW)])

    nself = jnp.where(wid < 31, _SLW, K - 31 * _SLW)

    @pl.when(wid < 31)
    def _():
        pltpu.sync_copy(wexp_h.at[pl.ds(E + wid * _SLW, _SLW)],
                        wwin.at[pl.ds(0, _SLW)])

    @pl.when(wid == 31)
    def _():
        pltpu.sync_copy(wexp_h.at[pl.ds(E + 31 * _SLW, 64)],
                        wwin.at[pl.ds(0, 64)])

    @pl.loop(0, _SLW // 16)
    def _(c):
        @pl.when(c * 16 < nself)
        def _():
            poff = pl.multiple_of(wid * _SLW + c * 16, 16)
            coff = pl.multiple_of(c * 16, 16)
            p16 = permb[pl.ds(poff, 16)]
            we = wwin[pl.ds(coff, 16)]
            dsum = plsc.load_gather(wsbuf, [p16])
            den = jnp.where(dsum > 0, dsum, jnp.ones((16,), jnp.float32))
            owin[pl.ds(coff, 16)] = we / den

    @pl.when(wid < 31)
    def _():
        pltpu.sync_copy(owin.at[pl.ds(0, _SLW)],
                        out_h.at[pl.ds(E + wid * _SLW, _SLW)])

    @pl.when(wid == 31)
    def _():
        pltpu.sync_copy(owin.at[pl.ds(0, 64)],
                        out_h.at[pl.ds(E + 31 * _SLW, 64)])


def _edge2_sc(dst, wexp_all, wsumf, perm):
    mesh = plsc.VectorSubcoreMesh(core_axis_name="c", subcore_axis_name="s")
    return pl.kernel(
        _edge2_sc_kernel,
        out_type=jax.ShapeDtypeStruct((E + K,), jnp.float32),
        mesh=mesh,
        compiler_params=_sc_compiler_params(),
        scratch_types=[
            pltpu.VMEM((N,), jnp.float32),
            pltpu.VMEM((K,), jnp.int32),
            pltpu.VMEM((_TW,), jnp.int32),
            pltpu.VMEM((_TW,), jnp.float32),
            pltpu.VMEM((_TW,), jnp.float32),
            pltpu.SemaphoreType.DMA,
        ],
    )(dst, wexp_all, wsumf, perm)


def _worker_params(in_deg_counts):
    r_incl = jnp.cumsum(in_deg_counts.astype(jnp.int32))
    r_excl = r_incl - in_deg_counts.astype(jnp.int32)
    b = jnp.asarray(_BOUNDS)
    blo = b[:32]
    bhi = b[1:33]
    vlo = jnp.searchsorted(r_incl, blo, side="right").astype(jnp.int32)
    vhi = jnp.searchsorted(r_incl, bhi - 1, side="right").astype(jnp.int32)
    skip_lo = blo - r_excl[vlo]
    take_hi = bhi - r_excl[vhi]
    zeros = jnp.zeros((32,), jnp.int32)
    return jnp.stack([vlo, vhi, skip_lo.astype(jnp.int32),
                      take_hi.astype(jnp.int32)] + [zeros] * 12, axis=1)


def kernel(feat, edge_index, e_feat, W, a, att):
    src = edge_index[0]
    dst = edge_index[1]
    sparts, dparts = _deg_sc(src, dst)
    odc2, idc2 = pl.pallas_call(
        _isum_body,
        out_shape=[jax.ShapeDtypeStruct((1, N), jnp.int32),
                   jax.ShapeDtypeStruct((1, N), jnp.int32)],
    )(sparts, dparts)
    in_deg_counts = idc2[0].astype(jnp.float32)
    out_deg = jnp.maximum(odc2[0].astype(jnp.float32), 1.0)
    in_deg = jnp.maximum(in_deg_counts, 1.0)
    src_norm = jax.lax.rsqrt(out_deg)
    dst_norm = jax.lax.rsqrt(in_deg)

    node_attn2, feat_scaled, s_full2, t_full2 = pl.pallas_call(
        _attn_scale_body,
        grid=(N // _BLK,),
        in_specs=[
            pl.BlockSpec((_BLK, D), lambda i: (i, 0)),
            pl.BlockSpec((D, D), lambda i: (0, 0)),
            pl.BlockSpec((D, 1), lambda i: (0, 0)),
            pl.BlockSpec((_BLK, 1), lambda i: (i, 0)),
            pl.BlockSpec((D, 1), lambda i: (0, 0)),
            pl.BlockSpec((D, 1), lambda i: (0, 0)),
        ],
        out_specs=[
            pl.BlockSpec((_BLK, 1), lambda i: (i, 0)),
            pl.BlockSpec((_BLK, D), lambda i: (i, 0)),
            pl.BlockSpec((_BLK, 1), lambda i: (i, 0)),
            pl.BlockSpec((_BLK, 1), lambda i: (i, 0)),
        ],
        out_shape=[
            jax.ShapeDtypeStruct((N, 1), jnp.float32),
            jax.ShapeDtypeStruct((N, D), jnp.float32),
            jax.ShapeDtypeStruct((N, 1), jnp.float32),
            jax.ShapeDtypeStruct((N, 1), jnp.float32),
        ],
    )(feat, W, a, src_norm[:, None], att[0, :D][:, None], att[0, D:][:, None])

    wparams = _worker_params(in_deg_counts)
    parts = _agg_sc(dst, src, e_feat, feat_scaled, wparams)
    agg = _combine_tc(wparams[:, 0], parts)

    x_score2 = pl.pallas_call(
        _score_body,
        grid=(N // _BLK,),
        in_specs=[
            pl.BlockSpec((_BLK, D), lambda i: (i, 0)),
            pl.BlockSpec((_BLK, D), lambda i: (i, 0)),
            pl.BlockSpec((_BLK, 1), lambda i: (i, 0)),
            pl.BlockSpec((_BLK, 1), lambda i: (i, 0)),
        ],
        out_specs=pl.BlockSpec((_BLK, 1), lambda i: (i, 0)),
        out_shape=jax.ShapeDtypeStruct((N, 1), jnp.float32),
    )(feat, agg, dst_norm[:, None], node_attn2)
    x_score = x_score2[:, 0]

    _, perm = jax.lax.top_k(x_score, K)
    feat_p = _featp_sc(feat, perm)
    wexp_all, lsum_parts = _edge1_sc(src, dst, e_feat,
                                     s_full2[:, 0], t_full2[:, 0], perm)
    wsumf = pl.pallas_call(
        _wsum_body,
        out_shape=jax.ShapeDtypeStruct((1, N), jnp.float32),
    )(lsum_parts)
    weights = _edge2_sc(dst, wexp_all, wsumf[0], perm)
    return (feat_p, weights, perm, x_score)


# speedup vs baseline: 11.0367x; 1.0667x over previous
import dataclasses
import functools

import jax
import jax.numpy as jnp
import numpy as np
from jax import lax
from jax.experimental import pallas as pl
from jax.experimental.pallas import tpu as pltpu
from jax.experimental.pallas import tpu_sc as plsc

N = 10000
E = 320000
D = 128
RATIO = 0.8
LAMB = 1.0
NEG_SLOPE = 0.2
K = int(np.ceil(RATIO * N))

_BLK = 1000

_CHUNK_SIZES = ([10080] * 11 + [9840] * 4 + [9760]) * 2
_BOUNDS = np.concatenate([[0], np.cumsum(_CHUNK_SIZES)]).astype(np.int32)
assert _BOUNDS[-1] == E

_SCAN_W = 2000
_NSCAN = E // _SCAN_W
assert _NSCAN * _SCAN_W == E
_BATCH = 128
_NBATCH = 10240 // _BATCH
_ACC_ROWS = 512
_LIST_CAP = 10240
_HALF = 5008
_GARB = 64
_SH_ROWS = _HALF + _GARB


def _attn_scale_body(feat_ref, w_ref, a_ref, srcn_ref, attl_ref, attr_ref,
                     attn_ref, fs_ref, s_ref, t_ref):
    f = feat_ref[...]
    h = jnp.dot(f, w_ref[...])
    t = jnp.dot(h, a_ref[...])
    lr = jnp.where(t >= 0, t, t * NEG_SLOPE)
    attn_ref[...] = jax.nn.sigmoid(lr)
    fs_ref[...] = f * srcn_ref[...]
    s_ref[...] = jnp.dot(f, attl_ref[...])
    t_ref[...] = jnp.dot(f, attr_ref[...])


def _combine_body(vlo_ref, part_ref, out_ref, acc_ref):
    w = pl.program_id(0)

    @pl.when(w == 0)
    def _():
        acc_ref[...] = jnp.zeros_like(acc_ref)

    start = vlo_ref[w]
    acc_ref[pl.ds(start, _ACC_ROWS), :] = (
        acc_ref[pl.ds(start, _ACC_ROWS), :] + part_ref[0])

    @pl.when(w == 31)
    def _():
        out_ref[...] = acc_ref[pl.ds(0, N), :]


def _combine_tc(vlo32, parts):
    return pl.pallas_call(
        _combine_body,
        grid=(32,),
        in_specs=[
            pl.BlockSpec(memory_space=pltpu.SMEM),
            pl.BlockSpec((1, _ACC_ROWS, D), lambda w: (w, 0, 0)),
        ],
        out_specs=pl.BlockSpec((N, D), lambda w: (0, 0)),
        out_shape=jax.ShapeDtypeStruct((N, D), jnp.float32),
        scratch_shapes=[pltpu.VMEM((N + _ACC_ROWS, D), jnp.float32)],
    )(vlo32, parts)


def _score_body(feat_ref, agg_ref, dstn_ref, attn_ref, out_ref):
    fd = jnp.abs(feat_ref[...] - agg_ref[...] * dstn_ref[...])
    acc = fd[:, 0:8]
    for i in range(1, 16):
        acc = acc + fd[:, 8 * i:8 * i + 8]
    a4 = acc[:, 0:4] + acc[:, 4:8]
    a2 = a4[:, 0:2] + a4[:, 2:4]
    a1 = a2[:, 0:1] + a2[:, 1:2]
    out_ref[...] = a1 * attn_ref[...]


def _agg_sc_kernel(dst_h, src_h, ef_h, fs_h, wp_h, out_h,
                   dwin, ids, dvals, idsb, srcs, dstv, efv, ewm, rows,
                   idsb2, srcs2, dstv2, efv2, ewm2, rows2, acc,
                   wprm, sems):
    cid = lax.axis_index("c")
    sid = lax.axis_index("s")
    wid = sid * 2 + cid

    pltpu.sync_copy(wp_h.at[pl.ds(pl.multiple_of(wid * 16, 16), 16)], wprm)
    w16 = wprm[...]
    vlo = w16[0]
    vhi = w16[1]
    skip_lo = w16[2]
    take_hi = w16[3]

    z16 = jnp.zeros((16,), jnp.float32)

    @pl.loop(0, _ACC_ROWS)
    def _(r):
        for g in range(8):
            acc[r, pl.ds(g * 16, 16)] = z16

    zi16 = jnp.zeros((16,), jnp.int32)

    @pl.loop(0, _LIST_CAP // 16)
    def _(r):
        ids[pl.ds(r * 16, 16)] = zi16
        dvals[pl.ds(r * 16, 16)] = zi16

    lane = lax.iota(jnp.int32, 16)

    def scan_win(win, carry):
        cnt_lo, cnt_hi, n_own = carry
        pltpu.sync_copy(dst_h.at[pl.ds(pl.multiple_of(win * _SCAN_W, 16), _SCAN_W)], dwin)

        def scan_chunk(c, carry2):
            cnt_lo, cnt_hi, n_own = carry2
            one16 = jnp.ones((16,), jnp.int32)
            zero16 = jnp.zeros((16,), jnp.int32)
            d16 = dwin[pl.ds(pl.multiple_of(c * 16, 16), 16)]
            m_lo = d16 == vlo
            m_hi = d16 == vhi
            cs_lo = plsc.cumsum(jnp.where(m_lo, one16, zero16))
            cs_hi = plsc.cumsum(jnp.where(m_hi, one16, zero16))
            occ_lo = cnt_lo + cs_lo - 1
            occ_hi = cnt_hi + cs_hi - 1
            in_rng = (d16 >= vlo) & (d16 <= vhi)
            ok_lo = jnp.logical_not(m_lo) | (occ_lo >= skip_lo)
            ok_hi = jnp.logical_not(m_hi) | (occ_hi < take_hi)
            owned = in_rng & ok_lo & ok_hi
            cum = plsc.cumsum(jnp.where(owned, one16, zero16))
            addr = n_own + cum - 1
            eids = win * _SCAN_W + c * 16 + lane
            plsc.store_scatter(ids, [addr], eids, mask=owned)
            plsc.store_scatter(dvals, [addr], d16, mask=owned)
            return (cnt_lo + cs_lo[15], cnt_hi + cs_hi[15], n_own + cum[15])

        return lax.fori_loop(0, _SCAN_W // 16, scan_chunk, (cnt_lo, cnt_hi, n_own))

    _, _, n_own = lax.fori_loop(0, _NSCAN, scan_win,
                                (jnp.int32(0), jnp.int32(0), jnp.int32(0)))

    bufs = ((idsb, srcs, dstv, efv, ewm, rows),
            (idsb2, srcs2, dstv2, efv2, ewm2, rows2))

    def prepare(b, par):
        bidsb, bsrcs, bdstv, befv, _, brows = bufs[par]
        base = b * _BATCH
        for g in range(_BATCH // 16):
            off = pl.multiple_of(base + g * 16, 16)
            bidsb[pl.ds(g * 16, 16)] = ids[pl.ds(off, 16)]
            bdstv[pl.ds(g * 16, 16)] = dvals[pl.ds(off, 16)]
        pltpu.sync_copy(src_h.at[bidsb], bsrcs)
        pltpu.sync_copy(ef_h.at[bidsb], befv)
        pltpu.async_copy(fs_h.at[bsrcs], brows, sems.at[par])

    def consume(b, par):
        _, bsrcs, bdstv, befv, bewm, brows = bufs[par]
        nb = jnp.minimum(n_own - b * _BATCH, _BATCH)
        pltpu.make_async_copy(fs_h.at[bsrcs], brows, sems.at[par]).wait()
        for g in range(_BATCH // 16):
            sl = pl.ds(g * 16, 16)
            bewm[sl] = jnp.where(bsrcs[sl] == bdstv[sl], 0.0, befv[sl])

        @pl.loop(0, _BATCH // 16)
        def _(g):
            goff = pl.multiple_of(g * 16, 16)
            d16 = bdstv[pl.ds(goff, 16)]
            ew16 = bewm[pl.ds(goff, 16)]
            for l in range(16):
                j = g * 16 + l

                @pl.when(j < nb)
                def _():
                    s = jnp.minimum(d16[l] - vlo, _ACC_ROWS - 1)
                    w = ew16[l]
                    for seg in range(8):
                        sl = pl.ds(seg * 16, 16)
                        acc[s, sl] = acc[s, sl] + brows[j, sl] * w

    @pl.when(n_own > 0)
    def _():
        prepare(0, 0)

    def do_pair(g, _):
        b0 = g * 2
        b1 = b0 + 1

        @pl.when(b1 * _BATCH < n_own)
        def _():
            prepare(b1, 1)

        @pl.when(b0 * _BATCH < n_own)
        def _():
            consume(b0, 0)

        @pl.when((b0 + 2) * _BATCH < n_own)
        def _():
            prepare(b0 + 2, 0)

        @pl.when(b1 * _BATCH < n_own)
        def _():
            consume(b1, 1)

        return 0

    lax.fori_loop(0, _NBATCH // 2, do_pair, 0)

    pltpu.sync_copy(acc, out_h.at[wid])


def _sc_compiler_params():
    cp = pltpu.CompilerParams()
    if "needs_layout_passes" in pltpu.CompilerParams.__dataclass_fields__:
        cp = dataclasses.replace(cp, needs_layout_passes=False)
    return cp


def _agg_sc(dst, src, e_feat, feat_scaled, wparams):
    mesh = plsc.VectorSubcoreMesh(core_axis_name="c", subcore_axis_name="s")
    kern = pl.kernel(
        _agg_sc_kernel,
        out_type=jax.ShapeDtypeStruct((32, _ACC_ROWS, D), jnp.float32),
        mesh=mesh,
        compiler_params=_sc_compiler_params(),
        scratch_types=[
            pltpu.VMEM((_SCAN_W,), jnp.int32),
            pltpu.VMEM((_LIST_CAP,), jnp.int32),
            pltpu.VMEM((_LIST_CAP,), jnp.int32),
            pltpu.VMEM((_BATCH,), jnp.int32),
            pltpu.VMEM((_BATCH,), jnp.int32),
            pltpu.VMEM((_BATCH,), jnp.int32),
            pltpu.VMEM((_BATCH,), jnp.float32),
            pltpu.VMEM((_BATCH,), jnp.float32),
            pltpu.VMEM((_BATCH, D), jnp.float32),
            pltpu.VMEM((_BATCH,), jnp.int32),
            pltpu.VMEM((_BATCH,), jnp.int32),
            pltpu.VMEM((_BATCH,), jnp.int32),
            pltpu.VMEM((_BATCH,), jnp.float32),
            pltpu.VMEM((_BATCH,), jnp.float32),
            pltpu.VMEM((_BATCH, D), jnp.float32),
            pltpu.VMEM((_ACC_ROWS, D), jnp.float32),
            pltpu.VMEM((16,), jnp.int32),
            pltpu.SemaphoreType.DMA((2,)),
        ],
    )
    return kern(dst, src, e_feat, feat_scaled, wparams.reshape(-1))


def _deg_sc_kernel(src_h, dst_h, osrc_h, odst_h, hsrc, hdst, win):
    cid = lax.axis_index("c")
    sid = lax.axis_index("s")
    wid = sid * 2 + cid
    zi16 = jnp.zeros((16,), jnp.int32)

    @pl.loop(0, N // 16)
    def _(r):
        roff = pl.multiple_of(r * 16, 16)
        hsrc[pl.ds(roff, 16)] = zi16
        hdst[pl.ds(roff, 16)] = zi16

    for which, (in_h, hist) in enumerate([(src_h, hsrc), (dst_h, hdst)]):
        @pl.loop(0, (E // 32) // 2000)
        def _(w, in_h=in_h, hist=hist):
            base = pl.multiple_of(wid * (E // 32) + w * 2000, 16)
            pltpu.sync_copy(in_h.at[pl.ds(base, 2000)], win)

            @pl.loop(0, 2000 // 16)
            def _(c, hist=hist):
                coff = pl.multiple_of(c * 16, 16)
                v16 = win[pl.ds(coff, 16)]
                cnt, last = plsc.scan_count(v16)
                plsc.addupdate_scatter(hist, [v16], cnt, mask=last)

    pltpu.sync_copy(hsrc, osrc_h.at[wid])
    pltpu.sync_copy(hdst, odst_h.at[wid])


def _deg_sc(src, dst):
    mesh = plsc.VectorSubcoreMesh(core_axis_name="c", subcore_axis_name="s")
    return pl.kernel(
        _deg_sc_kernel,
        out_type=[
            jax.ShapeDtypeStruct((32, N), jnp.int32),
            jax.ShapeDtypeStruct((32, N), jnp.int32),
        ],
        mesh=mesh,
        compiler_params=_sc_compiler_params(),
        scratch_types=[
            pltpu.VMEM((N,), jnp.int32),
            pltpu.VMEM((N,), jnp.int32),
            pltpu.VMEM((2000,), jnp.int32),
        ],
    )(src, dst)


def _isum_body(p0_ref, p1_ref, o0_ref, o1_ref):
    o0_ref[...] = jnp.sum(p0_ref[...], axis=0, keepdims=True)
    o1_ref[...] = jnp.sum(p1_ref[...], axis=0, keepdims=True)


_EPW = E // 32
_SLW = 256


def _featp_sc_kernel(feat_h, perm_h, out_h, idxb, rows, sem):
    cid = lax.axis_index("c")
    sid = lax.axis_index("s")
    wid = sid * 2 + cid
    nrows = jnp.where(wid < 31, _SLW, K - 31 * _SLW)
    off = pl.multiple_of(wid * _SLW, 8)

    @pl.when(wid < 31)
    def _():
        pltpu.sync_copy(perm_h.at[pl.ds(off, _SLW)], idxb)
        c1 = pltpu.async_copy(feat_h.at[idxb.at[pl.ds(0, 128)]],
                              rows.at[pl.ds(0, 128)], sem)
        c2 = pltpu.async_copy(feat_h.at[idxb.at[pl.ds(128, 128)]],
                              rows.at[pl.ds(128, 128)], sem)
        c1.wait()
        c2.wait()
        pltpu.sync_copy(rows, out_h.at[pl.ds(off, _SLW)])

    @pl.when(wid == 31)
    def _():
        pltpu.sync_copy(perm_h.at[pl.ds(31 * _SLW, 64)], idxb.at[pl.ds(0, 64)])
        pltpu.async_copy(feat_h.at[idxb.at[pl.ds(0, 64)]],
                         rows.at[pl.ds(0, 64)], sem).wait()
        pltpu.sync_copy(rows.at[pl.ds(0, 64)], out_h.at[pl.ds(31 * _SLW, 64)])


def _featp_sc(feat, perm):
    mesh = plsc.VectorSubcoreMesh(core_axis_name="c", subcore_axis_name="s")
    return pl.kernel(
        _featp_sc_kernel,
        out_type=jax.ShapeDtypeStruct((K, D), jnp.float32),
        mesh=mesh,
        compiler_params=_sc_compiler_params(),
        scratch_types=[
            pltpu.VMEM((_SLW,), jnp.int32),
            pltpu.VMEM((_SLW, D), jnp.float32),
            pltpu.SemaphoreType.DMA,
        ],
    )(feat, perm)


_TW = 2000


def _lsum_rmw(lsum, d16, v16, lane):
    for l in range(16):
        d = d16[l]
        row16 = pl.multiple_of((d >> 4) * 16, 16)
        cur = lsum[pl.ds(row16, 16)]
        add = jnp.where(lane == (d & 15), v16[l], 0.0)
        lsum[pl.ds(row16, 16)] = cur + add


def _edge1_sc_kernel(src_h, dst_h, ef_h, s_h, t_h, perm_h,
                     wexp_h, lsum_h,
                     sbuf, tbuf, sel, lsum, permb, swin, dwin, efwin, wwin,
                     sem):
    cid = lax.axis_index("c")
    sid = lax.axis_index("s")
    wid = sid * 2 + cid
    lane = lax.iota(jnp.int32, 16)
    zf16 = jnp.zeros((16,), jnp.float32)
    zi16 = jnp.zeros((16,), jnp.int32)
    one16 = jnp.full((16,), 1, jnp.int32)

    pltpu.sync_copy(s_h, sbuf)
    pltpu.sync_copy(t_h, tbuf)
    pltpu.sync_copy(perm_h, permb)

    @pl.loop(0, N // 16)
    def _(r):
        roff = pl.multiple_of(r * 16, 16)
        sel[pl.ds(roff, 16)] = zi16
        lsum[pl.ds(roff, 16)] = zf16

    @pl.loop(0, K // 16)
    def _(r):
        roff = pl.multiple_of(r * 16, 16)
        p16 = permb[pl.ds(roff, 16)]
        plsc.store_scatter(sel, [p16], one16)

    nself = jnp.where(wid < 31, _SLW, K - 31 * _SLW)

    @pl.loop(0, _SLW // 16)
    def _(c):
        @pl.when(c * 16 < nself)
        def _():
            poff = pl.multiple_of(wid * _SLW + c * 16, 16)
            p16 = permb[pl.ds(poff, 16)]
            sv = plsc.load_gather(sbuf, [p16])
            tv = plsc.load_gather(tbuf, [p16])
            x = sv + tv
            wraw = jnp.where(x >= 0, x, x * NEG_SLOPE) + 1.0
            we = jnp.exp(wraw)
            wwin[pl.ds(pl.multiple_of(c * 16, 16), 16)] = we
            _lsum_rmw(lsum, p16, we, lane)

    @pl.when(wid < 31)
    def _():
        pltpu.sync_copy(wwin.at[pl.ds(0, _SLW)],
                        wexp_h.at[pl.ds(E + wid * _SLW, _SLW)])

    @pl.when(wid == 31)
    def _():
        pltpu.sync_copy(wwin.at[pl.ds(0, 64)],
                        wexp_h.at[pl.ds(E + 31 * _SLW, 64)])

    @pl.loop(0, _EPW // _TW)
    def _(win):
        base = pl.multiple_of(wid * _EPW + win * _TW, 16)
        pltpu.sync_copy(src_h.at[pl.ds(base, _TW)], swin)
        pltpu.sync_copy(dst_h.at[pl.ds(base, _TW)], dwin)
        pltpu.sync_copy(ef_h.at[pl.ds(base, _TW)], efwin)

        @pl.loop(0, _TW // 16)
        def _(c):
            coff = pl.multiple_of(c * 16, 16)
            s16 = swin[pl.ds(coff, 16)]
            d16 = dwin[pl.ds(coff, 16)]
            e16 = efwin[pl.ds(coff, 16)]
            sv = plsc.load_gather(sbuf, [s16])
            tv = plsc.load_gather(tbuf, [d16])
            vs = plsc.load_gather(sel, [s16])
            vd = plsc.load_gather(sel, [d16])
            valid = (vs > 0) & (vd > 0)
            x = sv + tv
            wraw = jnp.where(x >= 0, x, x * NEG_SLOPE) + e16 * LAMB
            we = jnp.where(valid, jnp.exp(wraw), zf16)
            wwin[pl.ds(coff, 16)] = we
            _lsum_rmw(lsum, d16, we, lane)

        pltpu.sync_copy(wwin.at[pl.ds(0, _TW)], wexp_h.at[pl.ds(base, _TW)])

    pltpu.sync_copy(lsum, lsum_h.at[wid])


def _edge1_sc(src, dst, e_feat, s_full, t_full, perm):
    mesh = plsc.VectorSubcoreMesh(core_axis_name="c", subcore_axis_name="s")
    return pl.kernel(
        _edge1_sc_kernel,
        out_type=[
            jax.ShapeDtypeStruct((E + K,), jnp.float32),
            jax.ShapeDtypeStruct((32, N), jnp.float32),
        ],
        mesh=mesh,
        compiler_params=_sc_compiler_params(),
        scratch_types=[
            pltpu.VMEM((N,), jnp.float32),
            pltpu.VMEM((N,), jnp.float32),
            pltpu.VMEM((N,), jnp.int32),
            pltpu.VMEM((N,), jnp.float32),
            pltpu.VMEM((K,), jnp.int32),
            pltpu.VMEM((_TW,), jnp.int32),
            pltpu.VMEM((_TW,), jnp.int32),
            pltpu.VMEM((_TW,), jnp.float32),
            pltpu.VMEM((_TW,), jnp.float32),
            pltpu.SemaphoreType.DMA,
        ],
    )(src, dst, e_feat, s_full, t_full, perm)


def _wsum_body(parts_ref, out_ref):
    out_ref[...] = jnp.sum(parts_ref[...], axis=0, keepdims=True)


def _edge2_sc_kernel(dst_h, wexp_h, wsum_h, perm_h, out_h,
                     wsbuf, permb, dwin, wwin, owin, sem):
    cid = lax.axis_index("c")
    sid = lax.axis_index("s")
    wid = sid * 2 + cid

    pltpu.sync_copy(wsum_h, wsbuf)
    pltpu.sync_copy(perm_h, permb)

    @pl.loop(0, _EPW // _TW)
    def _(win):
        base = pl.multiple_of(wid * _EPW + win * _TW, 16)
        pltpu.sync_copy(dst_h.at[pl.ds(base, _TW)], dwin)
        pltpu.sync_copy(wexp_h.at[pl.ds(base, _TW)], wwin)

        @pl.loop(0, _TW // 16)
        def _(c):
            coff = pl.multiple_of(c * 16, 16)
            d16 = dwin[pl.ds(coff, 16)]
            we = wwin[pl.ds(coff, 16)]
            dsum = plsc.load_gather(wsbuf, [d16])
            den = jnp.where(dsum > 0, dsum, jnp.ones((16,), jnp.float32))
            owin[pl.ds(coff, 16)] = we / den

        pltpu.sync_copy(owin.at[pl.ds(0, _TW)], out_h.at[pl.ds(base, _TW)])

    nself = jnp.where(wid < 31, _SLW, K - 31 * _SLW)

    @pl.when(wid < 31)
    def _():
        pltpu.sync_copy(wexp_h.at[pl.ds(E + wid * _SLW, _SLW)],
                        wwin.at[pl.ds(0, _SLW)])

    @pl.when(wid == 31)
    def _():
        pltpu.sync_copy(wexp_h.at[pl.ds(E + 31 * _SLW, 64)],
                        wwin.at[pl.ds(0, 64)])

    @pl.loop(0, _SLW // 16)
    def _(c):
        @pl.when(c * 16 < nself)
        def _():
            poff = pl.multiple_of(wid * _SLW + c * 16, 16)
            coff = pl.multiple_of(c * 16, 16)
            p16 = permb[pl.ds(poff, 16)]
            we = wwin[pl.ds(coff, 16)]
            dsum = plsc.load_gather(wsbuf, [p16])
            den = jnp.where(dsum > 0, dsum, jnp.ones((16,), jnp.float32))
            owin[pl.ds(coff, 16)] = we / den

    @pl.when(wid < 31)
    def _():
        pltpu.sync_copy(owin.at[pl.ds(0, _SLW)],
                        out_h.at[pl.ds(E + wid * _SLW, _SLW)])

    @pl.when(wid == 31)
    def _():
        pltpu.sync_copy(owin.at[pl.ds(0, 64)],
                        out_h.at[pl.ds(E + 31 * _SLW, 64)])


def _edge2_sc(dst, wexp_all, wsumf, perm):
    mesh = plsc.VectorSubcoreMesh(core_axis_name="c", subcore_axis_name="s")
    return pl.kernel(
        _edge2_sc_kernel,
        out_type=jax.ShapeDtypeStruct((E + K,), jnp.float32),
        mesh=mesh,
        compiler_params=_sc_compiler_params(),
        scratch_types=[
            pltpu.VMEM((N,), jnp.float32),
            pltpu.VMEM((K,), jnp.int32),
            pltpu.VMEM((_TW,), jnp.int32),
            pltpu.VMEM((_TW,), jnp.float32),
            pltpu.VMEM((_TW,), jnp.float32),
            pltpu.SemaphoreType.DMA,
        ],
    )(dst, wexp_all, wsumf, perm)


def _worker_params(in_deg_counts):
    r_incl = jnp.cumsum(in_deg_counts.astype(jnp.int32))
    r_excl = r_incl - in_deg_counts.astype(jnp.int32)
    b = jnp.asarray(_BOUNDS)
    blo = b[:32]
    bhi = b[1:33]
    vlo = jnp.searchsorted(r_incl, blo, side="right").astype(jnp.int32)
    vhi = jnp.searchsorted(r_incl, bhi - 1, side="right").astype(jnp.int32)
    skip_lo = blo - r_excl[vlo]
    take_hi = bhi - r_excl[vhi]
    zeros = jnp.zeros((32,), jnp.int32)
    return jnp.stack([vlo, vhi, skip_lo.astype(jnp.int32),
                      take_hi.astype(jnp.int32)] + [zeros] * 12, axis=1)


def kernel(feat, edge_index, e_feat, W, a, att):
    src = edge_index[0]
    dst = edge_index[1]
    sparts, dparts = _deg_sc(src, dst)
    odc2, idc2 = pl.pallas_call(
        _isum_body,
        out_shape=[jax.ShapeDtypeStruct((1, N), jnp.int32),
                   jax.ShapeDtypeStruct((1, N), jnp.int32)],
    )(sparts, dparts)
    in_deg_counts = idc2[0].astype(jnp.float32)
    out_deg = jnp.maximum(odc2[0].astype(jnp.float32), 1.0)
    in_deg = jnp.maximum(in_deg_counts, 1.0)
    src_norm = jax.lax.rsqrt(out_deg)
    dst_norm = jax.lax.rsqrt(in_deg)

    node_attn2, feat_scaled, s_full2, t_full2 = pl.pallas_call(
        _attn_scale_body,
        grid=(N // _BLK,),
        in_specs=[
            pl.BlockSpec((_BLK, D), lambda i: (i, 0)),
            pl.BlockSpec((D, D), lambda i: (0, 0)),
            pl.BlockSpec((D, 1), lambda i: (0, 0)),
            pl.BlockSpec((_BLK, 1), lambda i: (i, 0)),
            pl.BlockSpec((D, 1), lambda i: (0, 0)),
            pl.BlockSpec((D, 1), lambda i: (0, 0)),
        ],
        out_specs=[
            pl.BlockSpec((_BLK, 1), lambda i: (i, 0)),
            pl.BlockSpec((_BLK, D), lambda i: (i, 0)),
            pl.BlockSpec((_BLK, 1), lambda i: (i, 0)),
            pl.BlockSpec((_BLK, 1), lambda i: (i, 0)),
        ],
        out_shape=[
            jax.ShapeDtypeStruct((N, 1), jnp.float32),
            jax.ShapeDtypeStruct((N, D), jnp.float32),
            jax.ShapeDtypeStruct((N, 1), jnp.float32),
            jax.ShapeDtypeStruct((N, 1), jnp.float32),
        ],
    )(feat, W, a, src_norm[:, None], att[0, :D][:, None], att[0, D:][:, None])

    wparams = _worker_params(in_deg_counts)
    parts = _agg_sc(dst, src, e_feat, feat_scaled, wparams)
    agg = _combine_tc(wparams[:, 0], parts)

    x_score2 = pl.pallas_call(
        _score_body,
        grid=(N // _BLK,),
        in_specs=[
            pl.BlockSpec((_BLK, D), lambda i: (i, 0)),
            pl.BlockSpec((_BLK, D), lambda i: (i, 0)),
            pl.BlockSpec((_BLK, 1), lambda i: (i, 0)),
            pl.BlockSpec((_BLK, 1), lambda i: (i, 0)),
        ],
        out_specs=pl.BlockSpec((_BLK, 1), lambda i: (i, 0)),
        out_shape=jax.ShapeDtypeStruct((N, 1), jnp.float32),
    )(feat, agg, dst_norm[:, None], node_attn2)
    x_score = x_score2[:, 0]

    _, perm = jax.lax.top_k(x_score, K)
    feat_p = _featp_sc(feat, perm)
    wexp_all, lsum_parts = _edge1_sc(src, dst, e_feat,
                                     s_full2[:, 0], t_full2[:, 0], perm)
    wsumf = pl.pallas_call(
        _wsum_body,
        out_shape=jax.ShapeDtypeStruct((1, N), jnp.float32),
    )(lsum_parts)
    weights = _edge2_sc(dst, wexp_all, wsumf[0], perm)
    return (feat_p, weights, perm, x_score)
